# baseline (speedup 1.0000x reference)
import functools

import jax
import jax.numpy as jnp
from jax import lax
from jax.experimental import pallas as pl
from jax.experimental.pallas import tpu as pltpu
from jax.experimental.pallas import tpu_sc as plsc

NC = 2
NS = 16
LN = 16
KROW = 128
DUMMY = 96


def _sc_mesh():
    return plsc.VectorSubcoreMesh(
        core_axis_name="c", subcore_axis_name="s", num_cores=NC,
        num_subcores=NS)


def _make_deg_kernel(E, N):
    epw = E // (NC * NS)
    assert epw * NC * NS == E and epw % LN == 0

    @functools.partial(
        pl.kernel,
        out_type=jax.ShapeDtypeStruct((NC * NS, N), jnp.float32),
        mesh=_sc_mesh(),
        compiler_params=pltpu.CompilerParams(needs_layout_passes=False),
        scratch_types=[
            pltpu.VMEM((epw,), jnp.int32),
            pltpu.VMEM((N,), jnp.float32),
        ],
    )
    def deg_kernel(dst_hbm, out_hbm, dst_v, hist_v):
        w = lax.axis_index("c") * NS + lax.axis_index("s")
        pltpu.sync_copy(dst_hbm.at[w], dst_v)

        def zero_body(i, _):
            hist_v[pl.ds(i * LN, LN)] = jnp.zeros((LN,), jnp.float32)
            return 0
        lax.fori_loop(0, N // LN, zero_body, 0)

        ones = jnp.ones((LN,), jnp.float32)

        def count_body(i, _):
            idx = dst_v[pl.ds(i * LN, LN)]
            plsc.addupdate_scatter(hist_v, [idx], ones)
            return 0
        lax.fori_loop(0, epw // LN, count_body, 0)
        pltpu.sync_copy(hist_v, out_hbm.at[w])

    return deg_kernel


def _make_agg_kernel(T, N, H, G):
    tpc = T // NC
    rpw = ((N + DUMMY + NS * 8 - 1) // (NS * 8)) * 8
    n_pad = NS * rpw

    GB = 16
    assert G % GB == 0

    @functools.partial(
        pl.kernel,
        out_type=jax.ShapeDtypeStruct((T, n_pad, H), jnp.float32),
        mesh=_sc_mesh(),
        scratch_types=[
            pltpu.VMEM((2, GB, KROW), jnp.int32),
            pltpu.VMEM((2, GB, KROW), jnp.int32),
            pltpu.VMEM((2, KROW, H), jnp.float32),
            pltpu.VMEM_SHARED((n_pad, H), jnp.float32),
            pltpu.SemaphoreType.DMA((2,)),
            pltpu.SemaphoreType.DMA((2,)),
        ],
    )
    def agg_kernel(xw_hbm, srcp_hbm, dstp_hbm, agg_hbm,
                   src_v, dst_v, row_v, acc, sem, ssem):
        c = lax.axis_index("c")
        s = lax.axis_index("s")
        r0 = s * rpw

        def stage_idx(blk):
            slot = lax.rem(blk, 2)
            pltpu.sync_copy(srcp_hbm.at[s, pl.ds(blk * GB, GB)],
                            src_v.at[slot])
            pltpu.sync_copy(dstp_hbm.at[s, pl.ds(blk * GB, GB)],
                            dst_v.at[slot])

        def gather_desc(t, g):
            gs = lax.rem(g, 2)
            islot = lax.rem(g // GB, 2)
            within = lax.rem(g, GB)
            return pltpu.make_async_copy(
                xw_hbm.at[t].at[src_v.at[islot].at[within]],
                row_v.at[gs], sem.at[gs])

        def scatter_args(g):
            gs = lax.rem(g, 2)
            islot = lax.rem(g // GB, 2)
            within = lax.rem(g, GB)
            return (row_v.at[gs], acc.at[dst_v.at[islot].at[within]],
                    ssem.at[gs])

        def t_body(tt, _):
            t = c * tpc + tt
            pltpu.sync_copy(xw_hbm.at[t, pl.ds(r0, rpw)],
                            acc.at[pl.ds(r0, rpw)])
            plsc.subcore_barrier()
            stage_idx(0)
            gather_desc(t, 0).start()

            def g_body(g, _):
                nb = g + 1

                @pl.when(nb < G)
                def _():
                    @pl.when(g >= 1)
                    def _():
                        sr, dr, sm = scatter_args(g - 1)
                        pltpu.make_async_copy(sr, dr, sm).wait()

                    @pl.when(lax.rem(nb, GB) == 0)
                    def _():
                        stage_idx(nb // GB)
                    gather_desc(t, nb).start()

                gather_desc(t, g).wait()
                sr, dr, sm = scatter_args(g)
                pltpu.async_copy(sr, dr, sm, add=True)
                return 0
            lax.fori_loop(0, G, g_body, 0)
            sr, dr, sm = scatter_args(G - 1)
            pltpu.make_async_copy(sr, dr, sm).wait()
            sr, dr, sm = scatter_args(G - 2)
            pltpu.make_async_copy(sr, dr, sm).wait()
            plsc.subcore_barrier()
            pltpu.sync_copy(acc.at[pl.ds(r0, rpw)],
                            agg_hbm.at[t, pl.ds(r0, rpw)])
            plsc.subcore_barrier()
            return 0
        lax.fori_loop(0, tpc, t_body, 0)

    return agg_kernel


def _scale_body(x_ref, degp_ref, w_ref, xw_ref, dis_ref):
    b = pl.program_id(0)
    deg = 1.0 + jnp.sum(degp_ref[...], axis=1, keepdims=True)
    dis = jnp.where(b == 0, lax.rsqrt(deg), 1.0)
    x = x_ref[0, 0]
    xw = jnp.dot(x, w_ref[...], preferred_element_type=jnp.float32)
    xw_ref[0] = xw * dis
    dis_ref[...] = dis


def _head_body(agg_ref, xw_ref, dis_ref, gcnb_ref, wih_ref, whh_ref,
               bih_ref, bhh_ref, linw_ref, linb_ref, out_ref, *, T, H, JB):
    j = pl.program_id(0)
    is_b0 = j < JB
    rows = out_ref.shape[0]
    dis = dis_ref[...]
    gcnb = gcnb_ref[...]
    wih = wih_ref[...]
    whh = whh_ref[...]
    bih = bih_ref[...]
    bhh = bhh_ref[...]
    dn = (((1,), (1,)), ((), ()))
    h = jnp.zeros((rows, H), jnp.float32)
    for t in range(T):
        a = jnp.where(is_b0, agg_ref[t], xw_ref[t])
        x_t = a * dis + gcnb
        gi = lax.dot_general(x_t, wih, dn,
                             preferred_element_type=jnp.float32) + bih
        gh = lax.dot_general(h, whh, dn,
                             preferred_element_type=jnp.float32) + bhh
        r = jax.nn.sigmoid(gi[:, :H] + gh[:, :H])
        z = jax.nn.sigmoid(gi[:, H:2 * H] + gh[:, H:2 * H])
        n_ = jnp.tanh(gi[:, 2 * H:] + r * gh[:, 2 * H:])
        h = (1.0 - z) * n_ + z * h
    out_ref[...] = (jnp.dot(h, linw_ref[...],
                            preferred_element_type=jnp.float32)
                    + linb_ref[...])


def kernel(x_seq, edge_index, edge_weight, gcn_W, gcn_b, W_ih, W_hh,
           b_ih, b_hh, lin_W, lin_b):
    del edge_weight
    B, T, N, F = x_seq.shape
    H = gcn_W.shape[1]
    P = lin_W.shape[1]
    E = edge_index.shape[1]
    ei = edge_index.astype(jnp.int32)
    src, dst = ei[0], ei[1]

    dst_a = dst.reshape(NC * NS, E // (NC * NS))
    gblk = NS * KROW * 16
    e_pad = ((E + gblk - 1) // gblk) * gblk
    npad = e_pad - E
    pad_src = (jnp.arange(npad, dtype=jnp.int32) * 911) % N
    pad_dst = N + jnp.arange(npad, dtype=jnp.int32) % DUMMY
    G = e_pad // (NS * KROW)
    srcp = jnp.concatenate([src, pad_src]).reshape(NS, G, KROW)
    dstp = jnp.concatenate([dst, pad_dst]).reshape(NS, G, KROW)

    deg_parts = _make_deg_kernel(E, N)(dst_a)
    degp_t = jnp.transpose(deg_parts)

    RC = 400
    JB = N // RC
    grid_c = (B, T, JB)
    xw, dis_col = pl.pallas_call(
        _scale_body,
        grid=grid_c,
        in_specs=[
            pl.BlockSpec((1, 1, RC, F), lambda b, t, j: (b, t, j, 0)),
            pl.BlockSpec((RC, NC * NS), lambda b, t, j: (j, 0)),
            pl.BlockSpec((F, H), lambda b, t, j: (0, 0)),
        ],
        out_specs=[
            pl.BlockSpec((1, RC, H), lambda b, t, j: (t, b * JB + j, 0)),
            pl.BlockSpec((RC, 1), lambda b, t, j: (b * JB + j, 0)),
        ],
        out_shape=[
            jax.ShapeDtypeStruct((T, B * N, H), jnp.float32),
            jax.ShapeDtypeStruct((B * N, 1), jnp.float32),
        ],
    )(x_seq, degp_t, gcn_W)

    agg = _make_agg_kernel(T, N, H, G)(xw, srcp, dstp)

    PP = 8
    lin_Wp = jnp.pad(lin_W, ((0, 0), (0, PP - P)))
    lin_bp = jnp.pad(lin_b, (0, PP - P)).reshape(1, PP)
    RD = 400
    JD = B * N // RD
    out = pl.pallas_call(
        functools.partial(_head_body, T=T, H=H, JB=JB),
        grid=(JD,),
        in_specs=[
            pl.BlockSpec((T, RD, H), lambda j: (0, jnp.minimum(j, JB - 1), 0)),
            pl.BlockSpec((T, RD, H), lambda j: (0, j, 0)),
            pl.BlockSpec((RD, 1), lambda j: (j, 0)),
            pl.BlockSpec((1, H), lambda j: (0, 0)),
            pl.BlockSpec((3 * H, H), lambda j: (0, 0)),
            pl.BlockSpec((3 * H, H), lambda j: (0, 0)),
            pl.BlockSpec((1, 3 * H), lambda j: (0, 0)),
            pl.BlockSpec((1, 3 * H), lambda j: (0, 0)),
            pl.BlockSpec((H, PP), lambda j: (0, 0)),
            pl.BlockSpec((1, PP), lambda j: (0, 0)),
        ],
        out_specs=pl.BlockSpec((RD, PP), lambda j: (j, 0)),
        out_shape=jax.ShapeDtypeStruct((B * N, PP), jnp.float32),
    )(agg, xw, dis_col, gcn_b.reshape(1, H), W_ih, W_hh,
      b_ih.reshape(1, 3 * H), b_hh.reshape(1, 3 * H), lin_Wp, lin_bp)

    return out[:, :P].reshape(B, N, P, 1).transpose(0, 2, 1, 3)


# speedup vs baseline: 1.0283x; 1.0283x over previous
import functools

import jax
import jax.numpy as jnp
from jax import lax
from jax.experimental import pallas as pl
from jax.experimental.pallas import tpu as pltpu
from jax.experimental.pallas import tpu_sc as plsc

NC = 2
NS = 16
LN = 16
KROW = 128
DUMMY = 96


def _sc_mesh():
    return plsc.VectorSubcoreMesh(
        core_axis_name="c", subcore_axis_name="s", num_cores=NC,
        num_subcores=NS)


def _make_deg_kernel(E, N):
    epw = E // (NC * NS)
    assert epw * NC * NS == E and epw % LN == 0

    @functools.partial(
        pl.kernel,
        out_type=jax.ShapeDtypeStruct((NC * NS, N), jnp.float32),
        mesh=_sc_mesh(),
        compiler_params=pltpu.CompilerParams(needs_layout_passes=False),
        scratch_types=[
            pltpu.VMEM((epw,), jnp.int32),
            pltpu.VMEM((N,), jnp.float32),
        ],
    )
    def deg_kernel(dst_hbm, out_hbm, dst_v, hist_v):
        w = lax.axis_index("c") * NS + lax.axis_index("s")
        pltpu.sync_copy(dst_hbm.at[w], dst_v)

        def zero_body(i, _):
            hist_v[pl.ds(i * LN, LN)] = jnp.zeros((LN,), jnp.float32)
            return 0
        lax.fori_loop(0, N // LN, zero_body, 0)

        ones = jnp.ones((LN,), jnp.float32)

        def count_body(i, _):
            idx = dst_v[pl.ds(i * LN, LN)]
            plsc.addupdate_scatter(hist_v, [idx], ones)
            return 0
        lax.fori_loop(0, epw // LN, count_body, 0)
        pltpu.sync_copy(hist_v, out_hbm.at[w])

    return deg_kernel


def _make_agg_kernel(T, N, H, G):
    tpc = T // NC
    rpw = ((N + DUMMY + NS * 8 - 1) // (NS * 8)) * 8
    n_pad = NS * rpw

    GB = 16
    assert G % GB == 0

    @functools.partial(
        pl.kernel,
        out_type=jax.ShapeDtypeStruct((T, n_pad, H), jnp.float32),
        mesh=_sc_mesh(),
        scratch_types=[
            pltpu.VMEM((2, GB, KROW), jnp.int32),
            pltpu.VMEM((2, GB, KROW), jnp.int32),
            pltpu.VMEM((2, KROW, H), jnp.float32),
            pltpu.VMEM_SHARED((n_pad, H), jnp.float32),
            pltpu.SemaphoreType.DMA((2,)),
            pltpu.SemaphoreType.DMA((2,)),
        ],
    )
    def agg_kernel(xw_hbm, srcp_hbm, dstp_hbm, agg_hbm,
                   src_v, dst_v, row_v, acc, sem, ssem):
        c = lax.axis_index("c")
        s = lax.axis_index("s")
        r0 = s * rpw

        def stage_idx(blk):
            slot = lax.rem(blk, 2)
            pltpu.sync_copy(srcp_hbm.at[s, pl.ds(blk * GB, GB)],
                            src_v.at[slot])
            pltpu.sync_copy(dstp_hbm.at[s, pl.ds(blk * GB, GB)],
                            dst_v.at[slot])

        def gather_desc(t, g):
            gs = lax.rem(g, 2)
            islot = lax.rem(g // GB, 2)
            within = lax.rem(g, GB)
            return pltpu.make_async_copy(
                xw_hbm.at[t].at[src_v.at[islot].at[within]],
                row_v.at[gs], sem.at[gs])

        def scatter_args(g):
            gs = lax.rem(g, 2)
            islot = lax.rem(g // GB, 2)
            within = lax.rem(g, GB)
            return (row_v.at[gs], acc.at[dst_v.at[islot].at[within]],
                    ssem.at[gs])

        def t_body(tt, _):
            t = c * tpc + tt
            pltpu.sync_copy(xw_hbm.at[t, pl.ds(r0, rpw)],
                            acc.at[pl.ds(r0, rpw)])
            plsc.subcore_barrier()
            stage_idx(0)
            gather_desc(t, 0).start()

            def g_body(g, _):
                nb = g + 1

                @pl.when(nb < G)
                def _():
                    @pl.when(g >= 1)
                    def _():
                        sr, dr, sm = scatter_args(g - 1)
                        pltpu.make_async_copy(sr, dr, sm).wait()

                    @pl.when(lax.rem(nb, GB) == 0)
                    def _():
                        stage_idx(nb // GB)
                    gather_desc(t, nb).start()

                gather_desc(t, g).wait()
                sr, dr, sm = scatter_args(g)
                pltpu.async_copy(sr, dr, sm, add=True)
                return 0
            lax.fori_loop(0, G, g_body, 0)
            sr, dr, sm = scatter_args(G - 1)
            pltpu.make_async_copy(sr, dr, sm).wait()
            sr, dr, sm = scatter_args(G - 2)
            pltpu.make_async_copy(sr, dr, sm).wait()
            plsc.subcore_barrier()
            pltpu.sync_copy(acc.at[pl.ds(r0, rpw)],
                            agg_hbm.at[t, pl.ds(r0, rpw)])
            plsc.subcore_barrier()
            return 0
        lax.fori_loop(0, tpc, t_body, 0)

    return agg_kernel


def _scale_body(x_ref, degp_ref, w_ref, xw_ref, dis_ref):
    b = pl.program_id(0)
    deg = 1.0 + jnp.sum(degp_ref[...], axis=1, keepdims=True)
    dis = jnp.where(b == 0, lax.rsqrt(deg), 1.0)
    x = x_ref[0, 0]
    xw = jnp.dot(x, w_ref[...], preferred_element_type=jnp.float32)
    xw_ref[0] = xw * dis
    dis_ref[...] = dis


def _head_body(data_ref, dis_ref, gcnb_ref, wih_ref, whh_ref,
               bih_ref, bhh_ref, linw_ref, linb_ref, out_ref, *, T, H):
    rows = out_ref.shape[0]
    dis = dis_ref[...]
    gcnb = gcnb_ref[...]
    wih = wih_ref[...]
    whh = whh_ref[...]
    bih = bih_ref[...]
    bhh = bhh_ref[...]
    dn = (((1,), (1,)), ((), ()))
    h = jnp.zeros((rows, H), jnp.float32)
    for t in range(T):
        x_t = data_ref[t] * dis + gcnb
        gi = lax.dot_general(x_t.astype(jnp.bfloat16), wih, dn,
                             preferred_element_type=jnp.float32) + bih
        gh = lax.dot_general(h.astype(jnp.bfloat16), whh, dn,
                             preferred_element_type=jnp.float32) + bhh
        r = jax.nn.sigmoid(gi[:, :H] + gh[:, :H])
        z = jax.nn.sigmoid(gi[:, H:2 * H] + gh[:, H:2 * H])
        n_ = jnp.tanh(gi[:, 2 * H:] + r * gh[:, 2 * H:])
        h = (1.0 - z) * n_ + z * h
    out_ref[...] = (jnp.dot(h, linw_ref[...],
                            preferred_element_type=jnp.float32)
                    + linb_ref[...])


def kernel(x_seq, edge_index, edge_weight, gcn_W, gcn_b, W_ih, W_hh,
           b_ih, b_hh, lin_W, lin_b):
    del edge_weight
    B, T, N, F = x_seq.shape
    H = gcn_W.shape[1]
    P = lin_W.shape[1]
    E = edge_index.shape[1]
    ei = edge_index.astype(jnp.int32)
    src, dst = ei[0], ei[1]

    dst_a = dst.reshape(NC * NS, E // (NC * NS))
    gblk = NS * KROW * 16
    e_pad = ((E + gblk - 1) // gblk) * gblk
    npad = e_pad - E
    pad_src = (jnp.arange(npad, dtype=jnp.int32) * 911) % N
    pad_dst = N + jnp.arange(npad, dtype=jnp.int32) % DUMMY
    G = e_pad // (NS * KROW)
    srcp = jnp.concatenate([src, pad_src]).reshape(NS, G, KROW)
    dstp = jnp.concatenate([dst, pad_dst]).reshape(NS, G, KROW)

    deg_parts = _make_deg_kernel(E, N)(dst_a)
    degp_t = jnp.transpose(deg_parts)

    RC = 400
    JB = N // RC
    grid_c = (B, T, JB)
    xw, dis_col = pl.pallas_call(
        _scale_body,
        grid=grid_c,
        in_specs=[
            pl.BlockSpec((1, 1, RC, F), lambda b, t, j: (b, t, j, 0)),
            pl.BlockSpec((RC, NC * NS), lambda b, t, j: (j, 0)),
            pl.BlockSpec((F, H), lambda b, t, j: (0, 0)),
        ],
        out_specs=[
            pl.BlockSpec((1, RC, H), lambda b, t, j: (t, b * JB + j, 0)),
            pl.BlockSpec((RC, 1), lambda b, t, j: (b * JB + j, 0)),
        ],
        out_shape=[
            jax.ShapeDtypeStruct((T, B * N, H), jnp.float32),
            jax.ShapeDtypeStruct((B * N, 1), jnp.float32),
        ],
    )(x_seq, degp_t, gcn_W)

    agg = _make_agg_kernel(T, N, H, G)(xw, srcp, dstp)

    PP = 8
    lin_Wp = jnp.pad(lin_W, ((0, 0), (0, PP - P)))
    lin_bp = jnp.pad(lin_b, (0, PP - P)).reshape(1, PP)
    wih_c = W_ih.astype(jnp.bfloat16)
    whh_c = W_hh.astype(jnp.bfloat16)
    RD = 400
    JD = N // RD
    scalars = (gcn_b.reshape(1, H), wih_c, whh_c, b_ih.reshape(1, 3 * H),
               b_hh.reshape(1, 3 * H), lin_Wp, lin_bp)
    scalar_specs = [
        pl.BlockSpec((1, H), lambda j: (0, 0)),
        pl.BlockSpec((3 * H, H), lambda j: (0, 0)),
        pl.BlockSpec((3 * H, H), lambda j: (0, 0)),
        pl.BlockSpec((1, 3 * H), lambda j: (0, 0)),
        pl.BlockSpec((1, 3 * H), lambda j: (0, 0)),
        pl.BlockSpec((H, PP), lambda j: (0, 0)),
        pl.BlockSpec((1, PP), lambda j: (0, 0)),
    ]
    body = functools.partial(_head_body, T=T, H=H)
    out0 = pl.pallas_call(
        body, grid=(JD,),
        in_specs=[
            pl.BlockSpec((T, RD, H), lambda j: (0, j, 0)),
            pl.BlockSpec((RD, 1), lambda j: (j, 0)),
        ] + scalar_specs,
        out_specs=pl.BlockSpec((RD, PP), lambda j: (j, 0)),
        out_shape=jax.ShapeDtypeStruct((N, PP), jnp.float32),
    )(agg, dis_col, *scalars)
    out1 = pl.pallas_call(
        body, grid=(JD,),
        in_specs=[
            pl.BlockSpec((T, RD, H), lambda j: (0, JD + j, 0)),
            pl.BlockSpec((RD, 1), lambda j: (JD + j, 0)),
        ] + scalar_specs,
        out_specs=pl.BlockSpec((RD, PP), lambda j: (j, 0)),
        out_shape=jax.ShapeDtypeStruct((N, PP), jnp.float32),
    )(xw, dis_col, *scalars)

    out = jnp.concatenate([out0[:, :P], out1[:, :P]], axis=0)
    return out.reshape(B, N, P, 1).transpose(0, 2, 1, 3)


# speedup vs baseline: 1.1377x; 1.1064x over previous
import functools

import jax
import jax.numpy as jnp
from jax import lax
from jax.experimental import pallas as pl
from jax.experimental.pallas import tpu as pltpu
from jax.experimental.pallas import tpu_sc as plsc

NC = 2
NS = 16
LN = 16
KROW = 128
DUMMY = 96


def _sc_mesh():
    return plsc.VectorSubcoreMesh(
        core_axis_name="c", subcore_axis_name="s", num_cores=NC,
        num_subcores=NS)


def _make_deg_kernel(E, N):
    epw = E // (NC * NS)
    assert epw * NC * NS == E and epw % LN == 0

    @functools.partial(
        pl.kernel,
        out_type=jax.ShapeDtypeStruct((NC * NS, N), jnp.float32),
        mesh=_sc_mesh(),
        compiler_params=pltpu.CompilerParams(needs_layout_passes=False),
        scratch_types=[
            pltpu.VMEM((epw,), jnp.int32),
            pltpu.VMEM((N,), jnp.float32),
        ],
    )
    def deg_kernel(dst_hbm, out_hbm, dst_v, hist_v):
        w = lax.axis_index("c") * NS + lax.axis_index("s")
        pltpu.sync_copy(dst_hbm.at[w], dst_v)

        def zero_body(i, _):
            hist_v[pl.ds(i * LN, LN)] = jnp.zeros((LN,), jnp.float32)
            return 0
        lax.fori_loop(0, N // LN, zero_body, 0)

        ones = jnp.ones((LN,), jnp.float32)

        def count_body(i, _):
            idx = dst_v[pl.ds(i * LN, LN)]
            plsc.addupdate_scatter(hist_v, [idx], ones)
            return 0
        lax.fori_loop(0, epw // LN, count_body, 0)
        pltpu.sync_copy(hist_v, out_hbm.at[w])

    return deg_kernel


def _make_agg_kernel(T, N, H, G):
    tpc = T // NC
    rpw = ((N + DUMMY + NS * 8 - 1) // (NS * 8)) * 8
    n_pad = NS * rpw

    GB = 16
    assert G % GB == 0

    @functools.partial(
        pl.kernel,
        out_type=jax.ShapeDtypeStruct((T, n_pad, H), jnp.float32),
        mesh=_sc_mesh(),
        scratch_types=[
            pltpu.VMEM((2, GB, KROW), jnp.int32),
            pltpu.VMEM((2, GB, KROW), jnp.int32),
            pltpu.VMEM((2, KROW, H), jnp.float32),
            pltpu.VMEM_SHARED((n_pad, H), jnp.float32),
            pltpu.SemaphoreType.DMA((2,)),
            pltpu.SemaphoreType.DMA((2,)),
        ],
    )
    def agg_kernel(xw_hbm, srcp_hbm, dstp_hbm, agg_hbm,
                   src_v, dst_v, row_v, acc, sem, ssem):
        c = lax.axis_index("c")
        s = lax.axis_index("s")
        r0 = s * rpw

        def stage_idx(blk):
            slot = lax.rem(blk, 2)
            pltpu.sync_copy(srcp_hbm.at[s, pl.ds(blk * GB, GB)],
                            src_v.at[slot])
            pltpu.sync_copy(dstp_hbm.at[s, pl.ds(blk * GB, GB)],
                            dst_v.at[slot])

        def gather_desc(t, g):
            gs = lax.rem(g, 2)
            islot = lax.rem(g // GB, 2)
            within = lax.rem(g, GB)
            return pltpu.make_async_copy(
                xw_hbm.at[t].at[src_v.at[islot].at[within]],
                row_v.at[gs], sem.at[gs])

        def scatter_args(g):
            gs = lax.rem(g, 2)
            islot = lax.rem(g // GB, 2)
            within = lax.rem(g, GB)
            return (row_v.at[gs], acc.at[dst_v.at[islot].at[within]],
                    ssem.at[gs])

        def t_body(tt, _):
            t = c * tpc + tt
            pltpu.sync_copy(xw_hbm.at[t, pl.ds(r0, rpw)],
                            acc.at[pl.ds(r0, rpw)])
            plsc.subcore_barrier()
            stage_idx(0)
            gather_desc(t, 0).start()

            def g_body(g, _):
                nb = g + 1

                @pl.when(nb < G)
                def _():
                    @pl.when(g >= 1)
                    def _():
                        sr, dr, sm = scatter_args(g - 1)
                        pltpu.make_async_copy(sr, dr, sm).wait()

                    @pl.when(lax.rem(nb, GB) == 0)
                    def _():
                        stage_idx(nb // GB)
                    gather_desc(t, nb).start()

                gather_desc(t, g).wait()
                sr, dr, sm = scatter_args(g)
                pltpu.async_copy(sr, dr, sm, add=True)
                return 0
            lax.fori_loop(0, G, g_body, 0)
            sr, dr, sm = scatter_args(G - 1)
            pltpu.make_async_copy(sr, dr, sm).wait()
            sr, dr, sm = scatter_args(G - 2)
            pltpu.make_async_copy(sr, dr, sm).wait()
            plsc.subcore_barrier()
            pltpu.sync_copy(acc.at[pl.ds(r0, rpw)],
                            agg_hbm.at[t, pl.ds(r0, rpw)])
            plsc.subcore_barrier()
            return 0
        lax.fori_loop(0, tpc, t_body, 0)

    return agg_kernel


def _scale_body(x_ref, degp_ref, w_ref, xw_ref, dis_ref):
    b = pl.program_id(0)
    deg = 1.0 + jnp.sum(degp_ref[...], axis=1, keepdims=True)
    dis = jnp.where(b == 0, lax.rsqrt(deg), 1.0)
    x = x_ref[0, 0]
    xw = jnp.dot(x, w_ref[...], preferred_element_type=jnp.float32)
    xw_ref[0] = xw * dis
    dis_ref[...] = dis


def _head_body(data_ref, dis_ref, gcnb_ref, wih_ref, whh_ref,
               bih_ref, bhh_ref, linw_ref, linb_ref, out_ref, *, T, H):
    rows = out_ref.shape[0]
    dis = dis_ref[...]
    gcnb = gcnb_ref[...]
    wih = wih_ref[...]
    whh = whh_ref[...]
    bih = bih_ref[...]
    bhh = bhh_ref[...]
    dn = (((1,), (1,)), ((), ()))
    h = jnp.zeros((rows, H), jnp.float32)
    for t in range(T):
        x_t = data_ref[t] * dis + gcnb
        gi = lax.dot_general(x_t.astype(jnp.bfloat16), wih, dn,
                             preferred_element_type=jnp.float32) + bih
        gh = lax.dot_general(h.astype(jnp.bfloat16), whh, dn,
                             preferred_element_type=jnp.float32) + bhh
        r = jax.nn.sigmoid(gi[:, :H] + gh[:, :H])
        z = jax.nn.sigmoid(gi[:, H:2 * H] + gh[:, H:2 * H])
        n_ = jnp.tanh(gi[:, 2 * H:] + r * gh[:, 2 * H:])
        h = (1.0 - z) * n_ + z * h
    out_ref[...] = (jnp.dot(h, linw_ref[...],
                            preferred_element_type=jnp.float32)
                    + linb_ref[...])


def kernel(x_seq, edge_index, edge_weight, gcn_W, gcn_b, W_ih, W_hh,
           b_ih, b_hh, lin_W, lin_b):
    del edge_weight
    B, T, N, F = x_seq.shape
    H = gcn_W.shape[1]
    P = lin_W.shape[1]
    E = edge_index.shape[1]
    ei = edge_index.astype(jnp.int32)
    src, dst = ei[0], ei[1]

    dst_a = dst.reshape(NC * NS, E // (NC * NS))
    gblk = NS * KROW * 16
    e_pad = ((E + gblk - 1) // gblk) * gblk
    npad = e_pad - E
    pad_src = (jnp.arange(npad, dtype=jnp.int32) * 911) % N
    pad_dst = N + jnp.arange(npad, dtype=jnp.int32) % DUMMY
    G = e_pad // (NS * KROW)
    srcp = jnp.concatenate([src, pad_src]).reshape(NS, G, KROW)
    dstp = jnp.concatenate([dst, pad_dst]).reshape(NS, G, KROW)

    deg_parts = _make_deg_kernel(E, N)(dst_a)
    degp_t = jnp.transpose(deg_parts)

    RC = 1000
    JB = N // RC
    grid_c = (B, T, JB)
    xw, dis_col = pl.pallas_call(
        _scale_body,
        grid=grid_c,
        in_specs=[
            pl.BlockSpec((1, 1, RC, F), lambda b, t, j: (b, t, j, 0)),
            pl.BlockSpec((RC, NC * NS), lambda b, t, j: (j, 0)),
            pl.BlockSpec((F, H), lambda b, t, j: (0, 0)),
        ],
        out_specs=[
            pl.BlockSpec((1, RC, H), lambda b, t, j: (t, b * JB + j, 0)),
            pl.BlockSpec((RC, 1), lambda b, t, j: (b * JB + j, 0)),
        ],
        out_shape=[
            jax.ShapeDtypeStruct((T, B * N, H), jnp.float32),
            jax.ShapeDtypeStruct((B * N, 1), jnp.float32),
        ],
    )(x_seq, degp_t, gcn_W)

    agg = _make_agg_kernel(T, N, H, G)(xw, srcp, dstp)

    PP = 8
    lin_Wp = jnp.pad(lin_W, ((0, 0), (0, PP - P)))
    lin_bp = jnp.pad(lin_b, (0, PP - P)).reshape(1, PP)
    wih_c = W_ih.astype(jnp.bfloat16)
    whh_c = W_hh.astype(jnp.bfloat16)
    RD = 1000
    JD = N // RD
    scalars = (gcn_b.reshape(1, H), wih_c, whh_c, b_ih.reshape(1, 3 * H),
               b_hh.reshape(1, 3 * H), lin_Wp, lin_bp)
    scalar_specs = [
        pl.BlockSpec((1, H), lambda j: (0, 0)),
        pl.BlockSpec((3 * H, H), lambda j: (0, 0)),
        pl.BlockSpec((3 * H, H), lambda j: (0, 0)),
        pl.BlockSpec((1, 3 * H), lambda j: (0, 0)),
        pl.BlockSpec((1, 3 * H), lambda j: (0, 0)),
        pl.BlockSpec((H, PP), lambda j: (0, 0)),
        pl.BlockSpec((1, PP), lambda j: (0, 0)),
    ]
    body = functools.partial(_head_body, T=T, H=H)
    out0 = pl.pallas_call(
        body, grid=(JD,),
        in_specs=[
            pl.BlockSpec((T, RD, H), lambda j: (0, j, 0)),
            pl.BlockSpec((RD, 1), lambda j: (j, 0)),
        ] + scalar_specs,
        out_specs=pl.BlockSpec((RD, PP), lambda j: (j, 0)),
        out_shape=jax.ShapeDtypeStruct((N, PP), jnp.float32),
    )(agg, dis_col, *scalars)
    out1 = pl.pallas_call(
        body, grid=(JD,),
        in_specs=[
            pl.BlockSpec((T, RD, H), lambda j: (0, JD + j, 0)),
            pl.BlockSpec((RD, 1), lambda j: (JD + j, 0)),
        ] + scalar_specs,
        out_specs=pl.BlockSpec((RD, PP), lambda j: (j, 0)),
        out_shape=jax.ShapeDtypeStruct((N, PP), jnp.float32),
    )(xw, dis_col, *scalars)

    out = jnp.concatenate([out0[:, :P], out1[:, :P]], axis=0)
    return out.reshape(B, N, P, 1).transpose(0, 2, 1, 3)


# speedup vs baseline: 1.1567x; 1.0167x over previous
import functools

import jax
import jax.numpy as jnp
from jax import lax
from jax.experimental import pallas as pl
from jax.experimental.pallas import tpu as pltpu
from jax.experimental.pallas import tpu_sc as plsc

NC = 2
NS = 16
LN = 16
KROW = 128
DUMMY = 96


def _sc_mesh():
    return plsc.VectorSubcoreMesh(
        core_axis_name="c", subcore_axis_name="s", num_cores=NC,
        num_subcores=NS)


def _make_deg_kernel(E, N):
    epw = E // (NC * NS)
    assert epw * NC * NS == E and epw % LN == 0

    @functools.partial(
        pl.kernel,
        out_type=jax.ShapeDtypeStruct((NC * NS, N), jnp.float32),
        mesh=_sc_mesh(),
        compiler_params=pltpu.CompilerParams(needs_layout_passes=False),
        scratch_types=[
            pltpu.VMEM((epw,), jnp.int32),
            pltpu.VMEM((N,), jnp.float32),
        ],
    )
    def deg_kernel(dst_hbm, out_hbm, dst_v, hist_v):
        w = lax.axis_index("c") * NS + lax.axis_index("s")
        pltpu.sync_copy(dst_hbm.at[w], dst_v)

        def zero_body(i, _):
            hist_v[pl.ds(i * LN, LN)] = jnp.zeros((LN,), jnp.float32)
            return 0
        lax.fori_loop(0, N // LN, zero_body, 0)

        ones = jnp.ones((LN,), jnp.float32)

        def count_body(i, _):
            idx = dst_v[pl.ds(i * LN, LN)]
            plsc.addupdate_scatter(hist_v, [idx], ones)
            return 0
        lax.fori_loop(0, epw // LN, count_body, 0)
        pltpu.sync_copy(hist_v, out_hbm.at[w])

    return deg_kernel


def _make_agg_kernel(T, N, H, G):
    tpc = T // NC
    rpw = ((N + DUMMY + NS * 8 - 1) // (NS * 8)) * 8
    n_pad = NS * rpw

    GB = 16
    assert G % GB == 0

    @functools.partial(
        pl.kernel,
        out_type=jax.ShapeDtypeStruct((T, n_pad, H), jnp.float32),
        mesh=_sc_mesh(),
        scratch_types=[
            pltpu.VMEM((2, GB, KROW), jnp.int32),
            pltpu.VMEM((2, GB, KROW), jnp.int32),
            pltpu.VMEM((2, KROW, H), jnp.float32),
            pltpu.VMEM_SHARED((n_pad, H), jnp.float32),
            pltpu.SemaphoreType.DMA((2,)),
            pltpu.SemaphoreType.DMA((2,)),
        ],
    )
    def agg_kernel(xw_hbm, srcp_hbm, dstp_hbm, agg_hbm,
                   src_v, dst_v, row_v, acc, sem, ssem):
        c = lax.axis_index("c")
        s = lax.axis_index("s")
        r0 = s * rpw

        def stage_idx(blk):
            slot = lax.rem(blk, 2)
            pltpu.sync_copy(srcp_hbm.at[s, pl.ds(blk * GB, GB)],
                            src_v.at[slot])
            pltpu.sync_copy(dstp_hbm.at[s, pl.ds(blk * GB, GB)],
                            dst_v.at[slot])

        def gather_desc(t, g):
            gs = lax.rem(g, 2)
            islot = lax.rem(g // GB, 2)
            within = lax.rem(g, GB)
            return pltpu.make_async_copy(
                xw_hbm.at[t].at[src_v.at[islot].at[within]],
                row_v.at[gs], sem.at[gs])

        def scatter_args(g):
            gs = lax.rem(g, 2)
            islot = lax.rem(g // GB, 2)
            within = lax.rem(g, GB)
            return (row_v.at[gs], acc.at[dst_v.at[islot].at[within]],
                    ssem.at[gs])

        pltpu.sync_copy(xw_hbm.at[c * tpc, pl.ds(r0, rpw)],
                        acc.at[pl.ds(r0, rpw)])
        plsc.subcore_barrier()

        def t_body(tt, _):
            t = c * tpc + tt
            stage_idx(0)
            gather_desc(t, 0).start()

            def g_body(g, _):
                nb = g + 1

                @pl.when(nb < G)
                def _():
                    @pl.when(g >= 1)
                    def _():
                        sr, dr, sm = scatter_args(g - 1)
                        pltpu.make_async_copy(sr, dr, sm).wait()

                    @pl.when(lax.rem(nb, GB) == 0)
                    def _():
                        stage_idx(nb // GB)
                    gather_desc(t, nb).start()

                gather_desc(t, g).wait()
                sr, dr, sm = scatter_args(g)
                pltpu.async_copy(sr, dr, sm, add=True)
                return 0
            lax.fori_loop(0, G, g_body, 0)
            sr, dr, sm = scatter_args(G - 1)
            pltpu.make_async_copy(sr, dr, sm).wait()
            sr, dr, sm = scatter_args(G - 2)
            pltpu.make_async_copy(sr, dr, sm).wait()
            plsc.subcore_barrier()
            pltpu.sync_copy(acc.at[pl.ds(r0, rpw)],
                            agg_hbm.at[t, pl.ds(r0, rpw)])

            @pl.when(tt + 1 < tpc)
            def _():
                pltpu.sync_copy(xw_hbm.at[t + 1, pl.ds(r0, rpw)],
                                acc.at[pl.ds(r0, rpw)])
            plsc.subcore_barrier()
            return 0
        lax.fori_loop(0, tpc, t_body, 0)

    return agg_kernel


def _scale_body(x_ref, degp_ref, w_ref, xw_ref, dis_ref):
    b = pl.program_id(0)
    deg = 1.0 + jnp.sum(degp_ref[...], axis=1, keepdims=True)
    dis = jnp.where(b == 0, lax.rsqrt(deg), 1.0)
    x = x_ref[0, 0]
    xw = jnp.dot(x, w_ref[...], preferred_element_type=jnp.float32)
    xw_ref[0] = xw * dis
    dis_ref[...] = dis


def _head_body(data_ref, dis_ref, gcnb_ref, wih_ref, whh_ref,
               bih_ref, bhh_ref, linw_ref, linb_ref, out_ref, *, T, H):
    rows = out_ref.shape[0]
    dis = dis_ref[...]
    gcnb = gcnb_ref[...]
    wih = wih_ref[...]
    whh = whh_ref[...]
    bih = bih_ref[...]
    bhh = bhh_ref[...]
    dn = (((1,), (1,)), ((), ()))
    h = jnp.zeros((rows, H), jnp.float32)
    for t in range(T):
        x_t = data_ref[t] * dis + gcnb
        gi = lax.dot_general(x_t.astype(jnp.bfloat16), wih, dn,
                             preferred_element_type=jnp.float32) + bih
        gh = lax.dot_general(h.astype(jnp.bfloat16), whh, dn,
                             preferred_element_type=jnp.float32) + bhh
        r = jax.nn.sigmoid(gi[:, :H] + gh[:, :H])
        z = jax.nn.sigmoid(gi[:, H:2 * H] + gh[:, H:2 * H])
        n_ = jnp.tanh(gi[:, 2 * H:] + r * gh[:, 2 * H:])
        h = (1.0 - z) * n_ + z * h
    out_ref[...] = (jnp.dot(h, linw_ref[...],
                            preferred_element_type=jnp.float32)
                    + linb_ref[...])


def kernel(x_seq, edge_index, edge_weight, gcn_W, gcn_b, W_ih, W_hh,
           b_ih, b_hh, lin_W, lin_b):
    del edge_weight
    B, T, N, F = x_seq.shape
    H = gcn_W.shape[1]
    P = lin_W.shape[1]
    E = edge_index.shape[1]
    ei = edge_index.astype(jnp.int32)
    src, dst = ei[0], ei[1]

    dst_a = dst.reshape(NC * NS, E // (NC * NS))
    gblk = NS * KROW * 16
    e_pad = ((E + gblk - 1) // gblk) * gblk
    npad = e_pad - E
    pad_src = (jnp.arange(npad, dtype=jnp.int32) * 911) % N
    pad_dst = N + jnp.arange(npad, dtype=jnp.int32) % DUMMY
    G = e_pad // (NS * KROW)
    srcp = jnp.concatenate([src, pad_src]).reshape(NS, G, KROW)
    dstp = jnp.concatenate([dst, pad_dst]).reshape(NS, G, KROW)

    deg_parts = _make_deg_kernel(E, N)(dst_a)
    degp_t = jnp.transpose(deg_parts)

    RC = 1000
    JB = N // RC
    grid_c = (B, T, JB)
    xw, dis_col = pl.pallas_call(
        _scale_body,
        grid=grid_c,
        in_specs=[
            pl.BlockSpec((1, 1, RC, F), lambda b, t, j: (b, t, j, 0)),
            pl.BlockSpec((RC, NC * NS), lambda b, t, j: (j, 0)),
            pl.BlockSpec((F, H), lambda b, t, j: (0, 0)),
        ],
        out_specs=[
            pl.BlockSpec((1, RC, H), lambda b, t, j: (t, b * JB + j, 0)),
            pl.BlockSpec((RC, 1), lambda b, t, j: (b * JB + j, 0)),
        ],
        out_shape=[
            jax.ShapeDtypeStruct((T, B * N, H), jnp.float32),
            jax.ShapeDtypeStruct((B * N, 1), jnp.float32),
        ],
    )(x_seq, degp_t, gcn_W)

    agg = _make_agg_kernel(T, N, H, G)(xw, srcp, dstp)

    PP = 8
    lin_Wp = jnp.pad(lin_W, ((0, 0), (0, PP - P)))
    lin_bp = jnp.pad(lin_b, (0, PP - P)).reshape(1, PP)
    wih_c = W_ih.astype(jnp.bfloat16)
    whh_c = W_hh.astype(jnp.bfloat16)
    RD = 1000
    JD = N // RD
    scalars = (gcn_b.reshape(1, H), wih_c, whh_c, b_ih.reshape(1, 3 * H),
               b_hh.reshape(1, 3 * H), lin_Wp, lin_bp)
    scalar_specs = [
        pl.BlockSpec((1, H), lambda j: (0, 0)),
        pl.BlockSpec((3 * H, H), lambda j: (0, 0)),
        pl.BlockSpec((3 * H, H), lambda j: (0, 0)),
        pl.BlockSpec((1, 3 * H), lambda j: (0, 0)),
        pl.BlockSpec((1, 3 * H), lambda j: (0, 0)),
        pl.BlockSpec((H, PP), lambda j: (0, 0)),
        pl.BlockSpec((1, PP), lambda j: (0, 0)),
    ]
    body = functools.partial(_head_body, T=T, H=H)
    out0 = pl.pallas_call(
        body, grid=(JD,),
        in_specs=[
            pl.BlockSpec((T, RD, H), lambda j: (0, j, 0)),
            pl.BlockSpec((RD, 1), lambda j: (j, 0)),
        ] + scalar_specs,
        out_specs=pl.BlockSpec((RD, PP), lambda j: (j, 0)),
        out_shape=jax.ShapeDtypeStruct((N, PP), jnp.float32),
    )(agg, dis_col, *scalars)
    out1 = pl.pallas_call(
        body, grid=(JD,),
        in_specs=[
            pl.BlockSpec((T, RD, H), lambda j: (0, JD + j, 0)),
            pl.BlockSpec((RD, 1), lambda j: (JD + j, 0)),
        ] + scalar_specs,
        out_specs=pl.BlockSpec((RD, PP), lambda j: (j, 0)),
        out_shape=jax.ShapeDtypeStruct((N, PP), jnp.float32),
    )(xw, dis_col, *scalars)

    out = jnp.concatenate([out0[:, :P], out1[:, :P]], axis=0)
    return out.reshape(B, N, P, 1).transpose(0, 2, 1, 3)


# speedup vs baseline: 1.1928x; 1.0312x over previous
import functools

import jax
import jax.numpy as jnp
from jax import lax
from jax.experimental import pallas as pl
from jax.experimental.pallas import tpu as pltpu
from jax.experimental.pallas import tpu_sc as plsc

NC = 2
NS = 16
LN = 16
KROW = 128
DUMMY = 96


def _sc_mesh():
    return plsc.VectorSubcoreMesh(
        core_axis_name="c", subcore_axis_name="s", num_cores=NC,
        num_subcores=NS)


def _make_deg_kernel(E, N):
    epw = E // (NC * NS)
    assert epw * NC * NS == E and epw % LN == 0

    @functools.partial(
        pl.kernel,
        out_type=jax.ShapeDtypeStruct((NC * NS, N), jnp.float32),
        mesh=_sc_mesh(),
        compiler_params=pltpu.CompilerParams(needs_layout_passes=False),
        scratch_types=[
            pltpu.VMEM((epw,), jnp.int32),
            pltpu.VMEM((N,), jnp.float32),
        ],
    )
    def deg_kernel(dst_hbm, out_hbm, dst_v, hist_v):
        w = lax.axis_index("c") * NS + lax.axis_index("s")
        pltpu.sync_copy(dst_hbm.at[w], dst_v)

        def zero_body(i, _):
            hist_v[pl.ds(i * LN, LN)] = jnp.zeros((LN,), jnp.float32)
            return 0
        lax.fori_loop(0, N // LN, zero_body, 0)

        ones = jnp.ones((LN,), jnp.float32)

        def count_body(i, _):
            idx = dst_v[pl.ds(i * LN, LN)]
            plsc.addupdate_scatter(hist_v, [idx], ones)
            return 0
        lax.fori_loop(0, epw // LN, count_body, 0)
        pltpu.sync_copy(hist_v, out_hbm.at[w])

    return deg_kernel


def _make_agg_kernel(T, N, H, G):
    tpc = T // NC
    rpw = ((N + DUMMY + NS * 8 - 1) // (NS * 8)) * 8
    n_pad = NS * rpw

    GB = 16
    NBLK = G // GB
    assert G % GB == 0

    @functools.partial(
        pl.kernel,
        out_type=jax.ShapeDtypeStruct((T, n_pad, H), jnp.float32),
        mesh=_sc_mesh(),
        scratch_types=[
            pltpu.VMEM((2, GB, 2, KROW), jnp.int32),
            pltpu.VMEM((2, KROW, H), jnp.float32),
            pltpu.VMEM_SHARED((n_pad, H), jnp.float32),
            pltpu.SemaphoreType.DMA((2,)),
            pltpu.SemaphoreType.DMA((2,)),
            pltpu.SemaphoreType.DMA((2,)),
        ],
    )
    def agg_kernel(xw_hbm, sdp_hbm, agg_hbm,
                   sd_v, row_v, acc, sem, ssem, isem):
        c = lax.axis_index("c")
        s = lax.axis_index("s")
        r0 = s * rpw

        def idx_desc(blk):
            slot = lax.rem(blk, 2)
            return pltpu.make_async_copy(
                sdp_hbm.at[s, pl.ds(blk * GB, GB)], sd_v.at[slot],
                isem.at[slot])

        def gather_desc(t, g):
            gs = lax.rem(g, 2)
            islot = lax.rem(g // GB, 2)
            within = lax.rem(g, GB)
            return pltpu.make_async_copy(
                xw_hbm.at[t].at[sd_v.at[islot].at[within].at[0]],
                row_v.at[gs], sem.at[gs])

        def scatter_args(g):
            gs = lax.rem(g, 2)
            islot = lax.rem(g // GB, 2)
            within = lax.rem(g, GB)
            return (row_v.at[gs],
                    acc.at[sd_v.at[islot].at[within].at[1]],
                    ssem.at[gs])

        pltpu.sync_copy(xw_hbm.at[c * tpc, pl.ds(r0, rpw)],
                        acc.at[pl.ds(r0, rpw)])
        plsc.subcore_barrier()

        def t_body(tt, _):
            t = c * tpc + tt
            idx_desc(0).start()
            idx_desc(0).wait()
            if NBLK > 1:
                idx_desc(1).start()
            gather_desc(t, 0).start()

            def g_body(g, _):
                nb = g + 1

                @pl.when(nb < G)
                def _():
                    @pl.when(g >= 1)
                    def _():
                        sr, dr, sm = scatter_args(g - 1)
                        pltpu.make_async_copy(sr, dr, sm).wait()

                    @pl.when(lax.rem(nb, GB) == 0)
                    def _():
                        idx_desc(nb // GB).wait()

                    @pl.when(
                        jnp.logical_and(lax.rem(nb, GB) == 4,
                                        nb // GB + 1 < NBLK))
                    def _():
                        idx_desc(nb // GB + 1).start()

                    gather_desc(t, nb).start()

                gather_desc(t, g).wait()
                sr, dr, sm = scatter_args(g)
                pltpu.async_copy(sr, dr, sm, add=True)
                return 0
            lax.fori_loop(0, G, g_body, 0)
            sr, dr, sm = scatter_args(G - 1)
            pltpu.make_async_copy(sr, dr, sm).wait()
            sr, dr, sm = scatter_args(G - 2)
            pltpu.make_async_copy(sr, dr, sm).wait()
            plsc.subcore_barrier()
            pltpu.sync_copy(acc.at[pl.ds(r0, rpw)],
                            agg_hbm.at[t, pl.ds(r0, rpw)])

            @pl.when(tt + 1 < tpc)
            def _():
                pltpu.sync_copy(xw_hbm.at[t + 1, pl.ds(r0, rpw)],
                                acc.at[pl.ds(r0, rpw)])
            plsc.subcore_barrier()
            return 0
        lax.fori_loop(0, tpc, t_body, 0)

    return agg_kernel


def _scale_body(x_ref, degp_ref, w_ref, xw_ref, dis_ref):
    b = pl.program_id(0)
    deg = 1.0 + jnp.sum(degp_ref[...], axis=1, keepdims=True)
    dis = jnp.where(b == 0, lax.rsqrt(deg), 1.0)
    x = x_ref[0, 0]
    xw = jnp.dot(x, w_ref[...], preferred_element_type=jnp.float32)
    xw_ref[0] = xw * dis
    dis_ref[...] = dis


def _head_body(data_ref, dis_ref, gcnb_ref, wih_ref, whh_ref,
               bih_ref, bhh_ref, linw_ref, linb_ref, out_ref, *, T, H):
    rows = out_ref.shape[0]
    dis = dis_ref[...]
    gcnb = gcnb_ref[...]
    wih = wih_ref[...]
    whh = whh_ref[...]
    bih = bih_ref[...]
    bhh = bhh_ref[...]
    dn = (((1,), (1,)), ((), ()))
    h = jnp.zeros((rows, H), jnp.float32)
    for t in range(T):
        x_t = data_ref[t] * dis + gcnb
        gi = lax.dot_general(x_t.astype(jnp.bfloat16), wih, dn,
                             preferred_element_type=jnp.float32) + bih
        gh = lax.dot_general(h.astype(jnp.bfloat16), whh, dn,
                             preferred_element_type=jnp.float32) + bhh
        r = jax.nn.sigmoid(gi[:, :H] + gh[:, :H])
        z = jax.nn.sigmoid(gi[:, H:2 * H] + gh[:, H:2 * H])
        n_ = jnp.tanh(gi[:, 2 * H:] + r * gh[:, 2 * H:])
        h = (1.0 - z) * n_ + z * h
    out_ref[...] = (jnp.dot(h, linw_ref[...],
                            preferred_element_type=jnp.float32)
                    + linb_ref[...])


def kernel(x_seq, edge_index, edge_weight, gcn_W, gcn_b, W_ih, W_hh,
           b_ih, b_hh, lin_W, lin_b):
    del edge_weight
    B, T, N, F = x_seq.shape
    H = gcn_W.shape[1]
    P = lin_W.shape[1]
    E = edge_index.shape[1]
    ei = edge_index.astype(jnp.int32)
    src, dst = ei[0], ei[1]

    dst_a = dst.reshape(NC * NS, E // (NC * NS))
    gblk = NS * KROW * 16
    e_pad = ((E + gblk - 1) // gblk) * gblk
    npad = e_pad - E
    pad_src = (jnp.arange(npad, dtype=jnp.int32) * 911) % N
    pad_dst = N + jnp.arange(npad, dtype=jnp.int32) % DUMMY
    G = e_pad // (NS * KROW)
    srcp = jnp.concatenate([src, pad_src]).reshape(NS, G, KROW)
    dstp = jnp.concatenate([dst, pad_dst]).reshape(NS, G, KROW)
    sdp = jnp.stack([srcp, dstp], axis=2)

    deg_parts = _make_deg_kernel(E, N)(dst_a)
    degp_t = jnp.transpose(deg_parts)

    RC = 1000
    JB = N // RC
    grid_c = (B, T, JB)
    xw, dis_col = pl.pallas_call(
        _scale_body,
        grid=grid_c,
        in_specs=[
            pl.BlockSpec((1, 1, RC, F), lambda b, t, j: (b, t, j, 0)),
            pl.BlockSpec((RC, NC * NS), lambda b, t, j: (j, 0)),
            pl.BlockSpec((F, H), lambda b, t, j: (0, 0)),
        ],
        out_specs=[
            pl.BlockSpec((1, RC, H), lambda b, t, j: (t, b * JB + j, 0)),
            pl.BlockSpec((RC, 1), lambda b, t, j: (b * JB + j, 0)),
        ],
        out_shape=[
            jax.ShapeDtypeStruct((T, B * N, H), jnp.float32),
            jax.ShapeDtypeStruct((B * N, 1), jnp.float32),
        ],
    )(x_seq, degp_t, gcn_W)

    agg = _make_agg_kernel(T, N, H, G)(xw, sdp)

    PP = 8
    lin_Wp = jnp.pad(lin_W, ((0, 0), (0, PP - P)))
    lin_bp = jnp.pad(lin_b, (0, PP - P)).reshape(1, PP)
    wih_c = W_ih.astype(jnp.bfloat16)
    whh_c = W_hh.astype(jnp.bfloat16)
    RD = 1000
    JD = N // RD
    scalars = (gcn_b.reshape(1, H), wih_c, whh_c, b_ih.reshape(1, 3 * H),
               b_hh.reshape(1, 3 * H), lin_Wp, lin_bp)
    scalar_specs = [
        pl.BlockSpec((1, H), lambda j: (0, 0)),
        pl.BlockSpec((3 * H, H), lambda j: (0, 0)),
        pl.BlockSpec((3 * H, H), lambda j: (0, 0)),
        pl.BlockSpec((1, 3 * H), lambda j: (0, 0)),
        pl.BlockSpec((1, 3 * H), lambda j: (0, 0)),
        pl.BlockSpec((H, PP), lambda j: (0, 0)),
        pl.BlockSpec((1, PP), lambda j: (0, 0)),
    ]
    body = functools.partial(_head_body, T=T, H=H)
    out0 = pl.pallas_call(
        body, grid=(JD,),
        in_specs=[
            pl.BlockSpec((T, RD, H), lambda j: (0, j, 0)),
            pl.BlockSpec((RD, 1), lambda j: (j, 0)),
        ] + scalar_specs,
        out_specs=pl.BlockSpec((RD, PP), lambda j: (j, 0)),
        out_shape=jax.ShapeDtypeStruct((N, PP), jnp.float32),
    )(agg, dis_col, *scalars)
    out1 = pl.pallas_call(
        body, grid=(JD,),
        in_specs=[
            pl.BlockSpec((T, RD, H), lambda j: (0, JD + j, 0)),
            pl.BlockSpec((RD, 1), lambda j: (JD + j, 0)),
        ] + scalar_specs,
        out_specs=pl.BlockSpec((RD, PP), lambda j: (j, 0)),
        out_shape=jax.ShapeDtypeStruct((N, PP), jnp.float32),
    )(xw, dis_col, *scalars)

    out = jnp.concatenate([out0[:, :P], out1[:, :P]], axis=0)
    return out.reshape(B, N, P, 1).transpose(0, 2, 1, 3)


# speedup vs baseline: 1.2025x; 1.0082x over previous
import functools

import jax
import jax.numpy as jnp
from jax import lax
from jax.experimental import pallas as pl
from jax.experimental.pallas import tpu as pltpu
from jax.experimental.pallas import tpu_sc as plsc

NC = 2
NS = 16
LN = 16
KROW = 128
DUMMY = 96


def _sc_mesh():
    return plsc.VectorSubcoreMesh(
        core_axis_name="c", subcore_axis_name="s", num_cores=NC,
        num_subcores=NS)


def _make_deg_kernel(E, N):
    epw = E // (NC * NS)
    assert epw * NC * NS == E and epw % LN == 0

    @functools.partial(
        pl.kernel,
        out_type=jax.ShapeDtypeStruct((NC * NS, N), jnp.float32),
        mesh=_sc_mesh(),
        compiler_params=pltpu.CompilerParams(needs_layout_passes=False),
        scratch_types=[
            pltpu.VMEM((epw,), jnp.int32),
            pltpu.VMEM((N,), jnp.float32),
        ],
    )
    def deg_kernel(dst_hbm, out_hbm, dst_v, hist_v):
        w = lax.axis_index("c") * NS + lax.axis_index("s")
        pltpu.sync_copy(dst_hbm.at[w], dst_v)

        def zero_body(i, _):
            hist_v[pl.ds(i * LN, LN)] = jnp.zeros((LN,), jnp.float32)
            return 0
        lax.fori_loop(0, N // LN, zero_body, 0)

        ones = jnp.ones((LN,), jnp.float32)

        def count_body(i, _):
            idx = dst_v[pl.ds(i * LN, LN)]
            plsc.addupdate_scatter(hist_v, [idx], ones)
            return 0
        lax.fori_loop(0, epw // LN, count_body, 0)
        pltpu.sync_copy(hist_v, out_hbm.at[w])

    return deg_kernel


def _make_agg_kernel(T, N, H, G):
    tpc = T // NC
    rpw = ((N + DUMMY + NS * 8 - 1) // (NS * 8)) * 8
    n_pad = NS * rpw

    GB = 16
    NBLK = G // GB
    assert G % GB == 0

    @functools.partial(
        pl.kernel,
        out_type=jax.ShapeDtypeStruct((T, n_pad, H), jnp.float32),
        mesh=_sc_mesh(),
        scratch_types=[
            pltpu.VMEM((2, GB, 2, KROW), jnp.int32),
            pltpu.VMEM((2, KROW, H), jnp.float32),
            pltpu.VMEM_SHARED((n_pad, H), jnp.float32),
            pltpu.SemaphoreType.DMA((2,)),
            pltpu.SemaphoreType.DMA((2,)),
            pltpu.SemaphoreType.DMA((2,)),
        ],
    )
    def agg_kernel(xw_hbm, sdp_hbm, agg_hbm,
                   sd_v, row_v, acc, sem, ssem, isem):
        c = lax.axis_index("c")
        s = lax.axis_index("s")
        r0 = s * rpw

        def idx_desc(blk):
            slot = lax.rem(blk, 2)
            return pltpu.make_async_copy(
                sdp_hbm.at[s, pl.ds(blk * GB, GB)], sd_v.at[slot],
                isem.at[slot])

        def gather_desc(t, g):
            gs = lax.rem(g, 2)
            islot = lax.rem(g // GB, 2)
            within = lax.rem(g, GB)
            return pltpu.make_async_copy(
                xw_hbm.at[t].at[sd_v.at[islot].at[within].at[0]],
                row_v.at[gs], sem.at[gs])

        def scatter_args(g):
            gs = lax.rem(g, 2)
            islot = lax.rem(g // GB, 2)
            within = lax.rem(g, GB)
            return (row_v.at[gs],
                    acc.at[sd_v.at[islot].at[within].at[1]],
                    ssem.at[gs])

        pltpu.sync_copy(xw_hbm.at[c * tpc, pl.ds(r0, rpw)],
                        acc.at[pl.ds(r0, rpw)])
        plsc.subcore_barrier()

        def t_body(tt, _):
            t = c * tpc + tt
            idx_desc(0).start()
            idx_desc(0).wait()
            if NBLK > 1:
                idx_desc(1).start()
            gather_desc(t, 0).start()

            def g_body(g, _):
                nb = g + 1

                @pl.when(nb < G)
                def _():
                    @pl.when(g >= 1)
                    def _():
                        sr, dr, sm = scatter_args(g - 1)
                        pltpu.make_async_copy(sr, dr, sm).wait()

                    @pl.when(lax.rem(nb, GB) == 0)
                    def _():
                        idx_desc(nb // GB).wait()

                    @pl.when(
                        jnp.logical_and(
                            lax.rem(nb, GB) == 4,
                            jnp.logical_and(nb // GB >= 1,
                                            nb // GB + 1 < NBLK)))
                    def _():
                        idx_desc(nb // GB + 1).start()

                    gather_desc(t, nb).start()

                gather_desc(t, g).wait()
                sr, dr, sm = scatter_args(g)
                pltpu.async_copy(sr, dr, sm, add=True)
                return 0
            lax.fori_loop(0, G, g_body, 0)
            sr, dr, sm = scatter_args(G - 1)
            pltpu.make_async_copy(sr, dr, sm).wait()
            sr, dr, sm = scatter_args(G - 2)
            pltpu.make_async_copy(sr, dr, sm).wait()
            plsc.subcore_barrier()
            pltpu.sync_copy(acc.at[pl.ds(r0, rpw)],
                            agg_hbm.at[t, pl.ds(r0, rpw)])

            @pl.when(tt + 1 < tpc)
            def _():
                pltpu.sync_copy(xw_hbm.at[t + 1, pl.ds(r0, rpw)],
                                acc.at[pl.ds(r0, rpw)])
            plsc.subcore_barrier()
            return 0
        lax.fori_loop(0, tpc, t_body, 0)

    return agg_kernel


def _scale_body(x_ref, degp_ref, w_ref, xw_ref, dis_ref):
    b = pl.program_id(0)
    deg = 1.0 + jnp.sum(degp_ref[...], axis=1, keepdims=True)
    dis = jnp.where(b == 0, lax.rsqrt(deg), 1.0)
    x = x_ref[0, 0]
    xw = jnp.dot(x, w_ref[...], preferred_element_type=jnp.float32)
    xw_ref[0] = xw * dis
    dis_ref[...] = dis


def _head_body(data_ref, dis_ref, gcnb_ref, wih_ref, whh_ref,
               bih_ref, bhh_ref, linw_ref, linb_ref, out_ref, *, T, H):
    rows = out_ref.shape[0]
    dis = dis_ref[...]
    gcnb = gcnb_ref[...]
    wih = wih_ref[...]
    whh = whh_ref[...]
    bih = bih_ref[...]
    bhh = bhh_ref[...]
    dn = (((1,), (1,)), ((), ()))
    h = jnp.zeros((rows, H), jnp.float32)
    for t in range(T):
        x_t = data_ref[t] * dis + gcnb
        gi = lax.dot_general(x_t.astype(jnp.bfloat16), wih, dn,
                             preferred_element_type=jnp.float32) + bih
        gh = lax.dot_general(h.astype(jnp.bfloat16), whh, dn,
                             preferred_element_type=jnp.float32) + bhh
        r = jax.nn.sigmoid(gi[:, :H] + gh[:, :H])
        z = jax.nn.sigmoid(gi[:, H:2 * H] + gh[:, H:2 * H])
        n_ = jnp.tanh(gi[:, 2 * H:] + r * gh[:, 2 * H:])
        h = (1.0 - z) * n_ + z * h
    out_ref[...] = (jnp.dot(h, linw_ref[...],
                            preferred_element_type=jnp.float32)
                    + linb_ref[...])


def kernel(x_seq, edge_index, edge_weight, gcn_W, gcn_b, W_ih, W_hh,
           b_ih, b_hh, lin_W, lin_b):
    del edge_weight
    B, T, N, F = x_seq.shape
    H = gcn_W.shape[1]
    P = lin_W.shape[1]
    E = edge_index.shape[1]
    ei = edge_index.astype(jnp.int32)
    src, dst = ei[0], ei[1]

    dst_a = dst.reshape(NC * NS, E // (NC * NS))
    gblk = NS * KROW * 16
    e_pad = ((E + gblk - 1) // gblk) * gblk
    npad = e_pad - E
    pad_src = (jnp.arange(npad, dtype=jnp.int32) * 911) % N
    pad_dst = N + jnp.arange(npad, dtype=jnp.int32) % DUMMY
    G = e_pad // (NS * KROW)
    srcp = jnp.concatenate([src, pad_src]).reshape(NS, G, KROW)
    dstp = jnp.concatenate([dst, pad_dst]).reshape(NS, G, KROW)
    sdp = jnp.stack([srcp, dstp], axis=2)

    deg_parts = _make_deg_kernel(E, N)(dst_a)
    degp_t = jnp.transpose(deg_parts)

    RC = 1000
    JB = N // RC
    grid_c = (B, T, JB)
    xw, dis_col = pl.pallas_call(
        _scale_body,
        grid=grid_c,
        in_specs=[
            pl.BlockSpec((1, 1, RC, F), lambda b, t, j: (b, t, j, 0)),
            pl.BlockSpec((RC, NC * NS), lambda b, t, j: (j, 0)),
            pl.BlockSpec((F, H), lambda b, t, j: (0, 0)),
        ],
        out_specs=[
            pl.BlockSpec((1, RC, H), lambda b, t, j: (t, b * JB + j, 0)),
            pl.BlockSpec((RC, 1), lambda b, t, j: (b * JB + j, 0)),
        ],
        out_shape=[
            jax.ShapeDtypeStruct((T, B * N, H), jnp.float32),
            jax.ShapeDtypeStruct((B * N, 1), jnp.float32),
        ],
    )(x_seq, degp_t, gcn_W)

    agg = _make_agg_kernel(T, N, H, G)(xw, sdp)

    PP = 8
    lin_Wp = jnp.pad(lin_W, ((0, 0), (0, PP - P)))
    lin_bp = jnp.pad(lin_b, (0, PP - P)).reshape(1, PP)
    wih_c = W_ih.astype(jnp.bfloat16)
    whh_c = W_hh.astype(jnp.bfloat16)
    RD = 1000
    JD = N // RD
    scalars = (gcn_b.reshape(1, H), wih_c, whh_c, b_ih.reshape(1, 3 * H),
               b_hh.reshape(1, 3 * H), lin_Wp, lin_bp)
    scalar_specs = [
        pl.BlockSpec((1, H), lambda j: (0, 0)),
        pl.BlockSpec((3 * H, H), lambda j: (0, 0)),
        pl.BlockSpec((3 * H, H), lambda j: (0, 0)),
        pl.BlockSpec((1, 3 * H), lambda j: (0, 0)),
        pl.BlockSpec((1, 3 * H), lambda j: (0, 0)),
        pl.BlockSpec((H, PP), lambda j: (0, 0)),
        pl.BlockSpec((1, PP), lambda j: (0, 0)),
    ]
    body = functools.partial(_head_body, T=T, H=H)
    out0 = pl.pallas_call(
        body, grid=(JD,),
        in_specs=[
            pl.BlockSpec((T, RD, H), lambda j: (0, j, 0)),
            pl.BlockSpec((RD, 1), lambda j: (j, 0)),
        ] + scalar_specs,
        out_specs=pl.BlockSpec((RD, PP), lambda j: (j, 0)),
        out_shape=jax.ShapeDtypeStruct((N, PP), jnp.float32),
    )(agg, dis_col, *scalars)
    out1 = pl.pallas_call(
        body, grid=(JD,),
        in_specs=[
            pl.BlockSpec((T, RD, H), lambda j: (0, JD + j, 0)),
            pl.BlockSpec((RD, 1), lambda j: (JD + j, 0)),
        ] + scalar_specs,
        out_specs=pl.BlockSpec((RD, PP), lambda j: (j, 0)),
        out_shape=jax.ShapeDtypeStruct((N, PP), jnp.float32),
    )(xw, dis_col, *scalars)

    out = jnp.concatenate([out0[:, :P], out1[:, :P]], axis=0)
    return out.reshape(B, N, P, 1).transpose(0, 2, 1, 3)
